# baseline (speedup 1.0000x reference)
import functools

import jax
import jax.numpy as jnp
from jax import lax
from jax.experimental import pallas as pl
from jax.experimental.pallas import tpu as pltpu
from jax.experimental.pallas import tpu_sc as plsc

_E = 8
_K = 2
_H = 1024
_I = 1024
_R = 16
_T = 2048
_BT = 256
_NB = (_T * _K) // _BT + _E
_P = _NB * _BT
_NW = 32
_SPT = _P // _NW
_TPT = _T // _NW
_GCH = 96
_CCH = 32

_MESH = dict(core_axis_name="c", subcore_axis_name="s", num_cores=2,
             num_subcores=16)


def _meta_body(ids_ref, tw_ref, dst_ref, tww_ref, be_ref):
    f32 = jnp.float32
    iota_e = lax.broadcasted_iota(jnp.int32, (1, _E), 1)
    rows = lax.broadcasted_iota(jnp.int32, (_BT, _BT), 0)
    cols = lax.broadcasted_iota(jnp.int32, (_BT, _BT), 1)
    tri = (rows > cols).astype(f32)
    dn = (((1,), (0,)), ((), ()))

    def scan_ranks(col):
        carry = jnp.zeros((1, _E), f32)
        oh_tiles, excl_tiles = [], []
        for i in range(_T // _BT):
            ids_i = ids_ref[pl.ds(i * _BT, _BT), col:col + 1]
            oh = (ids_i == iota_e).astype(f32)
            excl = lax.dot_general(tri, oh, dn,
                                   preferred_element_type=f32) + carry
            carry = carry + jnp.sum(oh, axis=0, keepdims=True)
            oh_tiles.append(oh)
            excl_tiles.append(excl)
        return (jnp.concatenate(oh_tiles, axis=0),
                jnp.concatenate(excl_tiles, axis=0), carry)

    oh0, excl0, tot0 = scan_ranks(0)
    oh1, excl1, tot1 = scan_ranks(1)
    counts = tot0 + tot1
    nblk = jnp.floor((counts + float(_BT - 1)) / float(_BT))
    lt = (lax.broadcasted_iota(jnp.int32, (_E, _E), 0) <
          lax.broadcasted_iota(jnp.int32, (_E, _E), 1)).astype(f32)
    blk_off = lax.dot_general(nblk, lt, dn, preferred_element_type=f32)
    pad_off = blk_off * float(_BT)

    dst0 = jnp.sum(oh0 * (pad_off + excl0), axis=1, keepdims=True)
    dst1 = jnp.sum(oh1 * (pad_off + tot0 + excl1), axis=1, keepdims=True)
    dst_ref[...] = jnp.concatenate([dst0, dst1], axis=1).astype(jnp.int32)

    tw = tw_ref[...]
    tww_ref[...] = tw / jnp.sum(tw, axis=1, keepdims=True)

    bi = lax.broadcasted_iota(jnp.int32, (_NB, _E), 0).astype(f32)
    ge = (bi >= jnp.broadcast_to(blk_off, (_NB, _E))).astype(f32)
    be_val = jnp.sum(ge, axis=1, keepdims=True) - 1.0
    total_blk = jnp.sum(nblk, axis=1, keepdims=True)
    active = bi[:, 0:1] < jnp.broadcast_to(total_blk, (_NB, 1))
    be_ref[...] = jnp.where(active, be_val, -1.0).astype(jnp.int32)


def _run_meta(topk_ids, topk_weights):
    return pl.pallas_call(
        _meta_body,
        out_shape=(
            jax.ShapeDtypeStruct((_T, _K), jnp.int32),
            jax.ShapeDtypeStruct((_T, _K), jnp.float32),
            jax.ShapeDtypeStruct((_NB, 1), jnp.int32),
        ),
    )(topk_ids, topk_weights)


@functools.lru_cache(maxsize=None)
def _make_dispatch():
    return functools.partial(
        pl.kernel,
        out_type=(
            jax.ShapeDtypeStruct((_P, _H), jnp.float32),
            jax.ShapeDtypeStruct((_P,), jnp.float32),
        ),
        mesh=plsc.VectorSubcoreMesh(**_MESH),
        compiler_params=pltpu.CompilerParams(needs_layout_passes=False),
        scratch_types=[
            pltpu.VMEM((_K, _T), jnp.int32),
            pltpu.VMEM((_K, _T), jnp.float32),
            pltpu.VMEM((_SPT,), jnp.int32),
            pltpu.VMEM((_SPT,), jnp.float32),
            pltpu.VMEM((_GCH, _H), jnp.float32),
            pltpu.SemaphoreType.DMA,
        ],
    )(_dispatch_body)


def _dispatch_body(dst_hbm, tww_hbm, hid_hbm, xs_hbm, sw_hbm,
                   dst_v, tww_v, tok_v, w_v, rows_v, sem):
    wid = lax.axis_index("s") * 2 + lax.axis_index("c")
    base = wid * _SPT
    pltpu.sync_copy(dst_hbm, dst_v)
    pltpu.sync_copy(tww_hbm, tww_v)

    zi = jnp.zeros((16,), jnp.int32)
    zf = jnp.zeros((16,), jnp.float32)
    for i in range(_SPT // 16):
        tok_v[pl.ds(i * 16, 16)] = zi
        w_v[pl.ds(i * 16, 16)] = zf

    lanes = lax.iota(jnp.int32, 16)

    def scatter_row(r):
        def body(c, _):
            idx = dst_v[r, pl.ds(c * 16, 16)]
            lidx = idx - base
            mask = (lidx >= 0) & (lidx < _SPT)
            lidx = jnp.where(mask, lidx, 0)
            toks = c * 16 + lanes
            plsc.store_scatter(tok_v, [lidx], toks, mask=mask)
            wv = tww_v[r, pl.ds(c * 16, 16)]
            plsc.store_scatter(w_v, [lidx], wv, mask=mask)
            return 0
        lax.fori_loop(0, _T // 16, body, 0)

    scatter_row(0)
    scatter_row(1)

    for i in range(_SPT // _GCH):
        pltpu.async_copy(hid_hbm.at[tok_v.at[pl.ds(i * _GCH, _GCH)]],
                         rows_v, sem).wait()
        pltpu.sync_copy(rows_v, xs_hbm.at[pl.ds(base + i * _GCH, _GCH)])
    pltpu.sync_copy(w_v, sw_hbm.at[pl.ds(base, _SPT)])


def _gmm_body(be_ref, xs_ref, sw_ref, w13_ref, w2_ref, gua_ref, gub_ref,
              da_ref, db_ref, out_ref):
    i = pl.program_id(0)
    active = be_ref[i] >= 0

    @pl.when(active)
    def _():
        x = xs_ref[...].astype(jnp.bfloat16)
        dn = (((1,), (1,)), ((), ()))
        gate_up = lax.dot_general(x, w13_ref[0], dn,
                                  preferred_element_type=jnp.float32)
        mid = lax.dot_general(x, gua_ref[0], dn,
                              preferred_element_type=jnp.float32)
        gate_up = gate_up + lax.dot_general(mid.astype(jnp.bfloat16),
                                            gub_ref[0], dn,
                                            preferred_element_type=jnp.float32)
        gate = gate_up[:, :_I]
        up = gate_up[:, _I:]
        act = (gate * jax.nn.sigmoid(gate) * up).astype(jnp.bfloat16)
        down = lax.dot_general(act, w2_ref[0], dn,
                               preferred_element_type=jnp.float32)
        dmid = lax.dot_general(act, da_ref[0], dn,
                               preferred_element_type=jnp.float32)
        down = down + lax.dot_general(dmid.astype(jnp.bfloat16), db_ref[0], dn,
                                      preferred_element_type=jnp.float32)
        out_ref[...] = sw_ref[...] * down

    @pl.when(jnp.logical_not(active))
    def _():
        out_ref[...] = jnp.zeros_like(out_ref)


def _run_gmm(be, xs, sw2d, w13, w2, gua, gub, da, db):
    def emap(i, be_r):
        return (jnp.where(be_r[i] < 0, _E - 1, be_r[i]), 0, 0)

    grid_spec = pltpu.PrefetchScalarGridSpec(
        num_scalar_prefetch=1,
        grid=(_NB,),
        in_specs=[
            pl.BlockSpec((_BT, _H), lambda i, be_r: (i, 0)),
            pl.BlockSpec((_BT, 1), lambda i, be_r: (i, 0)),
            pl.BlockSpec((1, 2 * _I, _H), emap),
            pl.BlockSpec((1, _H, _I), emap),
            pl.BlockSpec((1, _R, _H), emap),
            pl.BlockSpec((1, 2 * _I, _R), emap),
            pl.BlockSpec((1, _R, _I), emap),
            pl.BlockSpec((1, _H, _R), emap),
        ],
        out_specs=pl.BlockSpec((_BT, _H), lambda i, be_r: (i, 0)),
    )
    return pl.pallas_call(
        _gmm_body,
        grid_spec=grid_spec,
        out_shape=jax.ShapeDtypeStruct((_P, _H), jnp.float32),
    )(be, xs, sw2d, w13, w2, gua, gub, da, db)


@functools.lru_cache(maxsize=None)
def _make_combine():
    return functools.partial(
        pl.kernel,
        out_type=jax.ShapeDtypeStruct((_T, _H), jnp.float32),
        mesh=plsc.VectorSubcoreMesh(**_MESH),
        compiler_params=pltpu.CompilerParams(needs_layout_passes=False),
        scratch_types=[
            pltpu.VMEM((_TPT,), jnp.int32),
            pltpu.VMEM((_TPT,), jnp.int32),
            pltpu.VMEM((_CCH, _H), jnp.float32),
            pltpu.VMEM((_CCH, _H), jnp.float32),
            pltpu.SemaphoreType.DMA,
            pltpu.SemaphoreType.DMA,
        ],
    )(_combine_body)


def _combine_body(dstT_hbm, ys_hbm, out_hbm, idxA_v, idxB_v, bufA, bufB,
                  semA, semB):
    wid = lax.axis_index("s") * 2 + lax.axis_index("c")
    tbase = wid * _TPT
    pltpu.sync_copy(dstT_hbm.at[0, pl.ds(tbase, _TPT)], idxA_v)
    pltpu.sync_copy(dstT_hbm.at[1, pl.ds(tbase, _TPT)], idxB_v)

    for c2 in range(_TPT // _CCH):
        cpA = pltpu.async_copy(ys_hbm.at[idxA_v.at[pl.ds(c2 * _CCH, _CCH)]],
                               bufA, semA)
        cpB = pltpu.async_copy(ys_hbm.at[idxB_v.at[pl.ds(c2 * _CCH, _CCH)]],
                               bufB, semB)
        cpA.wait()
        cpB.wait()

        def add_body(j, _):
            r = j >> 6
            c = j & 63
            plsc.addupdate(bufA.at[r, pl.ds(c * 16, 16)],
                           bufB[r, pl.ds(c * 16, 16)])
            return 0
        lax.fori_loop(0, _CCH * (_H // 16), add_body, 0)
        pltpu.sync_copy(bufA, out_hbm.at[pl.ds(tbase + c2 * _CCH, _CCH)])


@jax.jit
def kernel(hidden_states, topk_weights, topk_ids, w13, w2, gate_up_lora_a,
           gate_up_lora_b, down_lora_a, down_lora_b):
    dst, tww, be = _run_meta(topk_ids, topk_weights)
    dst_t = dst.T
    tww_t = tww.T
    be24 = be.reshape(-1)
    xs, sw = _make_dispatch()(dst_t, tww_t, hidden_states)
    sw2d = sw.reshape(_P, 1)
    bf = jnp.bfloat16
    ys = _run_gmm(be24, xs, sw2d, w13.astype(bf), w2.astype(bf),
                  gate_up_lora_a.astype(bf), gate_up_lora_b.astype(bf),
                  down_lora_a.astype(bf), down_lora_b.astype(bf))
    return _make_combine()(dst_t, ys)


# speedup vs baseline: 1.1048x; 1.1048x over previous
import functools

import jax
import jax.numpy as jnp
from jax import lax
from jax.experimental import pallas as pl
from jax.experimental.pallas import tpu as pltpu
from jax.experimental.pallas import tpu_sc as plsc

_E = 8
_K = 2
_H = 1024
_I = 1024
_R = 16
_T = 2048
_BT = 256
_NB = (_T * _K) // _BT + _E
_P = _NB * _BT
_NW = 32
_SPT = _P // _NW
_TPT = _T // _NW
_GCH = 96
_CCH = 32

_MESH = dict(core_axis_name="c", subcore_axis_name="s", num_cores=2,
             num_subcores=16)


def _meta_body(ids_ref, tw_ref, dst_ref, tww_ref, be_ref):
    f32 = jnp.float32
    iota_e = lax.broadcasted_iota(jnp.int32, (1, _E), 1)
    rows = lax.broadcasted_iota(jnp.int32, (_BT, _BT), 0)
    cols = lax.broadcasted_iota(jnp.int32, (_BT, _BT), 1)
    tri = (rows > cols).astype(f32)
    dn = (((1,), (0,)), ((), ()))

    def scan_ranks(col):
        carry = jnp.zeros((1, _E), f32)
        oh_tiles, excl_tiles = [], []
        for i in range(_T // _BT):
            ids_i = ids_ref[pl.ds(i * _BT, _BT), col:col + 1]
            oh = (ids_i == iota_e).astype(f32)
            excl = lax.dot_general(tri, oh, dn,
                                   preferred_element_type=f32) + carry
            carry = carry + jnp.sum(oh, axis=0, keepdims=True)
            oh_tiles.append(oh)
            excl_tiles.append(excl)
        return (jnp.concatenate(oh_tiles, axis=0),
                jnp.concatenate(excl_tiles, axis=0), carry)

    oh0, excl0, tot0 = scan_ranks(0)
    oh1, excl1, tot1 = scan_ranks(1)
    counts = tot0 + tot1
    nblk = jnp.floor((counts + float(_BT - 1)) / float(_BT))
    lt = (lax.broadcasted_iota(jnp.int32, (_E, _E), 0) <
          lax.broadcasted_iota(jnp.int32, (_E, _E), 1)).astype(f32)
    blk_off = lax.dot_general(nblk, lt, dn, preferred_element_type=f32)
    pad_off = blk_off * float(_BT)

    dst0 = jnp.sum(oh0 * (pad_off + excl0), axis=1, keepdims=True)
    dst1 = jnp.sum(oh1 * (pad_off + tot0 + excl1), axis=1, keepdims=True)
    dst_ref[...] = jnp.concatenate([dst0, dst1], axis=1).astype(jnp.int32)

    tw = tw_ref[...]
    tww_ref[...] = tw / jnp.sum(tw, axis=1, keepdims=True)

    bi = lax.broadcasted_iota(jnp.int32, (_NB, _E), 0).astype(f32)
    ge = (bi >= jnp.broadcast_to(blk_off, (_NB, _E))).astype(f32)
    be_val = jnp.sum(ge, axis=1, keepdims=True) - 1.0
    total_blk = jnp.sum(nblk, axis=1, keepdims=True)
    active = bi[:, 0:1] < jnp.broadcast_to(total_blk, (_NB, 1))
    be_ref[...] = jnp.where(active, be_val, -1.0).astype(jnp.int32)


def _run_meta(topk_ids, topk_weights):
    return pl.pallas_call(
        _meta_body,
        out_shape=(
            jax.ShapeDtypeStruct((_T, _K), jnp.int32),
            jax.ShapeDtypeStruct((_T, _K), jnp.float32),
            jax.ShapeDtypeStruct((_NB, 1), jnp.int32),
        ),
    )(topk_ids, topk_weights)


@functools.lru_cache(maxsize=None)
def _make_dispatch():
    return functools.partial(
        pl.kernel,
        out_type=(
            jax.ShapeDtypeStruct((_P, _H), jnp.float32),
            jax.ShapeDtypeStruct((_P,), jnp.float32),
        ),
        mesh=plsc.VectorSubcoreMesh(**_MESH),
        compiler_params=pltpu.CompilerParams(needs_layout_passes=False),
        scratch_types=[
            pltpu.VMEM((_K, _T), jnp.int32),
            pltpu.VMEM((_K, _T), jnp.float32),
            pltpu.VMEM((_SPT,), jnp.int32),
            pltpu.VMEM((_SPT,), jnp.float32),
            pltpu.VMEM((_GCH, _H), jnp.float32),
            pltpu.SemaphoreType.DMA,
        ],
    )(_dispatch_body)


def _dispatch_body(dst_hbm, tww_hbm, hid_hbm, xs_hbm, sw_hbm,
                   dst_v, tww_v, tok_v, w_v, rows_v, sem):
    wid = lax.axis_index("s") * 2 + lax.axis_index("c")
    base = wid * _SPT
    pltpu.sync_copy(dst_hbm, dst_v)
    pltpu.sync_copy(tww_hbm, tww_v)

    zi = jnp.zeros((16,), jnp.int32)
    zf = jnp.zeros((16,), jnp.float32)
    for i in range(_SPT // 16):
        tok_v[pl.ds(i * 16, 16)] = zi
        w_v[pl.ds(i * 16, 16)] = zf

    lanes = lax.iota(jnp.int32, 16)

    def scatter_row(r):
        def body(c, _):
            idx = dst_v[r, pl.ds(c * 16, 16)]
            lidx = idx - base
            mask = (lidx >= 0) & (lidx < _SPT)
            lidx = jnp.where(mask, lidx, 0)
            toks = c * 16 + lanes
            plsc.store_scatter(tok_v, [lidx], toks, mask=mask)
            wv = tww_v[r, pl.ds(c * 16, 16)]
            plsc.store_scatter(w_v, [lidx], wv, mask=mask)
            return 0
        lax.fori_loop(0, _T // 16, body, 0)

    scatter_row(0)
    scatter_row(1)

    for i in range(_SPT // _GCH):
        pltpu.async_copy(hid_hbm.at[tok_v.at[pl.ds(i * _GCH, _GCH)]],
                         rows_v, sem).wait()
        pltpu.sync_copy(rows_v, xs_hbm.at[pl.ds(base + i * _GCH, _GCH)])
    pltpu.sync_copy(w_v, sw_hbm.at[pl.ds(base, _SPT)])


def _gmm_body(be_ref, xs_ref, sw_ref, w13_ref, w2_ref, gua_ref, gub_ref,
              da_ref, db_ref, out_ref):
    i = pl.program_id(0)
    active = be_ref[i] >= 0

    @pl.when(active)
    def _():
        bf = jnp.bfloat16
        x = xs_ref[...].astype(bf)
        dn = (((1,), (1,)), ((), ()))
        gate_up = lax.dot_general(x, w13_ref[0].astype(bf), dn,
                                  preferred_element_type=jnp.float32)
        mid = lax.dot_general(x, gua_ref[0].astype(bf), dn,
                              preferred_element_type=jnp.float32)
        gate_up = gate_up + lax.dot_general(mid.astype(bf),
                                            gub_ref[0].astype(bf), dn,
                                            preferred_element_type=jnp.float32)
        gate = gate_up[:, :_I]
        up = gate_up[:, _I:]
        act = (gate * jax.nn.sigmoid(gate) * up).astype(bf)
        down = lax.dot_general(act, w2_ref[0].astype(bf), dn,
                               preferred_element_type=jnp.float32)
        dmid = lax.dot_general(act, da_ref[0].astype(bf), dn,
                               preferred_element_type=jnp.float32)
        down = down + lax.dot_general(dmid.astype(bf), db_ref[0].astype(bf), dn,
                                      preferred_element_type=jnp.float32)
        out_ref[...] = sw_ref[...] * down

    @pl.when(jnp.logical_not(active))
    def _():
        out_ref[...] = jnp.zeros_like(out_ref)


def _run_gmm(be, xs, sw2d, w13, w2, gua, gub, da, db):
    def emap(i, be_r):
        return (jnp.where(be_r[i] < 0, _E - 1, be_r[i]), 0, 0)

    grid_spec = pltpu.PrefetchScalarGridSpec(
        num_scalar_prefetch=1,
        grid=(_NB,),
        in_specs=[
            pl.BlockSpec((_BT, _H), lambda i, be_r: (i, 0)),
            pl.BlockSpec((_BT, 1), lambda i, be_r: (i, 0)),
            pl.BlockSpec((1, 2 * _I, _H), emap),
            pl.BlockSpec((1, _H, _I), emap),
            pl.BlockSpec((1, _R, _H), emap),
            pl.BlockSpec((1, 2 * _I, _R), emap),
            pl.BlockSpec((1, _R, _I), emap),
            pl.BlockSpec((1, _H, _R), emap),
        ],
        out_specs=pl.BlockSpec((_BT, _H), lambda i, be_r: (i, 0)),
    )
    return pl.pallas_call(
        _gmm_body,
        grid_spec=grid_spec,
        out_shape=jax.ShapeDtypeStruct((_P, _H), jnp.float32),
    )(be, xs, sw2d, w13, w2, gua, gub, da, db)


@functools.lru_cache(maxsize=None)
def _make_combine():
    return functools.partial(
        pl.kernel,
        out_type=jax.ShapeDtypeStruct((_T, _H), jnp.float32),
        mesh=plsc.VectorSubcoreMesh(**_MESH),
        compiler_params=pltpu.CompilerParams(needs_layout_passes=False),
        scratch_types=[
            pltpu.VMEM((_TPT,), jnp.int32),
            pltpu.VMEM((_TPT,), jnp.int32),
            pltpu.VMEM((_CCH, _H), jnp.float32),
            pltpu.VMEM((_CCH, _H), jnp.float32),
            pltpu.SemaphoreType.DMA,
            pltpu.SemaphoreType.DMA,
        ],
    )(_combine_body)


def _combine_body(dstT_hbm, ys_hbm, out_hbm, idxA_v, idxB_v, bufA, bufB,
                  semA, semB):
    wid = lax.axis_index("s") * 2 + lax.axis_index("c")
    tbase = wid * _TPT
    pltpu.sync_copy(dstT_hbm.at[0, pl.ds(tbase, _TPT)], idxA_v)
    pltpu.sync_copy(dstT_hbm.at[1, pl.ds(tbase, _TPT)], idxB_v)

    for c2 in range(_TPT // _CCH):
        cpA = pltpu.async_copy(ys_hbm.at[idxA_v.at[pl.ds(c2 * _CCH, _CCH)]],
                               bufA, semA)
        cpB = pltpu.async_copy(ys_hbm.at[idxB_v.at[pl.ds(c2 * _CCH, _CCH)]],
                               bufB, semB)
        cpA.wait()
        cpB.wait()

        def add_body(j, _):
            r = j >> 6
            c = j & 63
            plsc.addupdate(bufA.at[r, pl.ds(c * 16, 16)],
                           bufB[r, pl.ds(c * 16, 16)])
            return 0
        lax.fori_loop(0, _CCH * (_H // 16), add_body, 0)
        pltpu.sync_copy(bufA, out_hbm.at[pl.ds(tbase + c2 * _CCH, _CCH)])


@jax.jit
def kernel(hidden_states, topk_weights, topk_ids, w13, w2, gate_up_lora_a,
           gate_up_lora_b, down_lora_a, down_lora_b):
    dst, tww, be = _run_meta(topk_ids, topk_weights)
    dst_t = dst.T
    tww_t = tww.T
    be24 = be.reshape(-1)
    xs, sw = _make_dispatch()(dst_t, tww_t, hidden_states)
    sw2d = sw.reshape(_P, 1)
    ys = _run_gmm(be24, xs, sw2d, w13, w2, gate_up_lora_a, gate_up_lora_b,
                  down_lora_a, down_lora_b)
    return _make_combine()(dst_t, ys)


# speedup vs baseline: 1.1100x; 1.0047x over previous
import functools

import jax
import jax.numpy as jnp
from jax import lax
from jax.experimental import pallas as pl
from jax.experimental.pallas import tpu as pltpu
from jax.experimental.pallas import tpu_sc as plsc

_E = 8
_K = 2
_H = 1024
_I = 1024
_R = 16
_T = 2048
_BT = 256
_NB = (_T * _K) // _BT + _E
_P = _NB * _BT
_NW = 32
_SPT = _P // _NW
_TPT = _T // _NW
_GCH = 48
_CCH = 32

_MESH = dict(core_axis_name="c", subcore_axis_name="s", num_cores=2,
             num_subcores=16)


def _meta_body(ids_ref, tw_ref, dst_ref, tww_ref, be_ref):
    f32 = jnp.float32
    iota_e = lax.broadcasted_iota(jnp.int32, (1, _E), 1)
    rows = lax.broadcasted_iota(jnp.int32, (_BT, _BT), 0)
    cols = lax.broadcasted_iota(jnp.int32, (_BT, _BT), 1)
    tri = (rows > cols).astype(f32)
    dn = (((1,), (0,)), ((), ()))

    def scan_ranks(col):
        carry = jnp.zeros((1, _E), f32)
        oh_tiles, excl_tiles = [], []
        for i in range(_T // _BT):
            ids_i = ids_ref[pl.ds(i * _BT, _BT), col:col + 1]
            oh = (ids_i == iota_e).astype(f32)
            excl = lax.dot_general(tri, oh, dn,
                                   preferred_element_type=f32) + carry
            carry = carry + jnp.sum(oh, axis=0, keepdims=True)
            oh_tiles.append(oh)
            excl_tiles.append(excl)
        return (jnp.concatenate(oh_tiles, axis=0),
                jnp.concatenate(excl_tiles, axis=0), carry)

    oh0, excl0, tot0 = scan_ranks(0)
    oh1, excl1, tot1 = scan_ranks(1)
    counts = tot0 + tot1
    nblk = jnp.floor((counts + float(_BT - 1)) / float(_BT))
    lt = (lax.broadcasted_iota(jnp.int32, (_E, _E), 0) <
          lax.broadcasted_iota(jnp.int32, (_E, _E), 1)).astype(f32)
    blk_off = lax.dot_general(nblk, lt, dn, preferred_element_type=f32)
    pad_off = blk_off * float(_BT)

    dst0 = jnp.sum(oh0 * (pad_off + excl0), axis=1, keepdims=True)
    dst1 = jnp.sum(oh1 * (pad_off + tot0 + excl1), axis=1, keepdims=True)
    dst_ref[...] = jnp.concatenate([dst0, dst1], axis=1).astype(jnp.int32)

    tw = tw_ref[...]
    tww_ref[...] = tw / jnp.sum(tw, axis=1, keepdims=True)

    bi = lax.broadcasted_iota(jnp.int32, (_NB, _E), 0).astype(f32)
    ge = (bi >= jnp.broadcast_to(blk_off, (_NB, _E))).astype(f32)
    be_val = jnp.sum(ge, axis=1, keepdims=True) - 1.0
    total_blk = jnp.sum(nblk, axis=1, keepdims=True)
    active = bi[:, 0:1] < jnp.broadcast_to(total_blk, (_NB, 1))
    be_ref[...] = jnp.where(active, be_val, -1.0).astype(jnp.int32)


def _run_meta(topk_ids, topk_weights):
    return pl.pallas_call(
        _meta_body,
        out_shape=(
            jax.ShapeDtypeStruct((_T, _K), jnp.int32),
            jax.ShapeDtypeStruct((_T, _K), jnp.float32),
            jax.ShapeDtypeStruct((_NB, 1), jnp.int32),
        ),
    )(topk_ids, topk_weights)


@functools.lru_cache(maxsize=None)
def _make_dispatch():
    return functools.partial(
        pl.kernel,
        out_type=(
            jax.ShapeDtypeStruct((_P, _H), jnp.float32),
            jax.ShapeDtypeStruct((_P,), jnp.float32),
        ),
        mesh=plsc.VectorSubcoreMesh(**_MESH),
        compiler_params=pltpu.CompilerParams(needs_layout_passes=False),
        scratch_types=[
            pltpu.VMEM((_K, _T), jnp.int32),
            pltpu.VMEM((_K, _T), jnp.float32),
            pltpu.VMEM((_SPT,), jnp.int32),
            pltpu.VMEM((_SPT,), jnp.float32),
            pltpu.VMEM((_GCH, _H), jnp.float32),
            pltpu.VMEM((_GCH, _H), jnp.float32),
            pltpu.SemaphoreType.DMA,
            pltpu.SemaphoreType.DMA,
        ],
    )(_dispatch_body)


def _dispatch_body(dst_hbm, tww_hbm, hid_hbm, xs_hbm, sw_hbm,
                   dst_v, tww_v, tok_v, w_v,
                   rows0, rows1, sem_g, sem_w):
    cid = lax.axis_index("c")
    sid = lax.axis_index("s")
    wid = sid * 2 + cid
    base = wid * _SPT

    pltpu.sync_copy(dst_hbm, dst_v)
    pltpu.sync_copy(tww_hbm, tww_v)

    zi = jnp.zeros((16,), jnp.int32)
    zf = jnp.zeros((16,), jnp.float32)
    for i in range(_SPT // 16):
        tok_v[pl.ds(i * 16, 16)] = zi
        w_v[pl.ds(i * 16, 16)] = zf

    lanes = lax.iota(jnp.int32, 16)

    def scatter_row(r):
        def body(c, _):
            idx = dst_v[r, pl.ds(c * 16, 16)]
            lidx = idx - base
            mask = (lidx >= 0) & (lidx < _SPT)
            lidx = jnp.where(mask, lidx, 0)
            toks = c * 16 + lanes
            plsc.store_scatter(tok_v, [lidx], toks, mask=mask)
            wv = tww_v[r, pl.ds(c * 16, 16)]
            plsc.store_scatter(w_v, [lidx], wv, mask=mask)
            return 0
        lax.fori_loop(0, _T // 16, body, 0)

    scatter_row(0)
    scatter_row(1)
    pltpu.sync_copy(w_v, sw_hbm.at[pl.ds(base, _SPT)])

    nch = _SPT // _GCH
    bufs = [rows0, rows1]
    gathers = [None] * nch
    writes = [None] * nch
    gathers[0] = pltpu.async_copy(hid_hbm.at[tok_v.at[pl.ds(0, _GCH)]],
                                  bufs[0], sem_g)
    if nch > 1:
        gathers[1] = pltpu.async_copy(hid_hbm.at[tok_v.at[pl.ds(_GCH, _GCH)]],
                                      bufs[1], sem_g)
    for i in range(nch):
        gathers[i].wait()
        writes[i] = pltpu.async_copy(
            bufs[i % 2], xs_hbm.at[pl.ds(base + i * _GCH, _GCH)], sem_w)
        if i + 2 < nch:
            writes[i].wait()
            gathers[i + 2] = pltpu.async_copy(
                hid_hbm.at[tok_v.at[pl.ds((i + 2) * _GCH, _GCH)]],
                bufs[i % 2], sem_g)
    for i in range(max(0, nch - 2), nch):
        writes[i].wait()


def _gmm_body(be_ref, xs_ref, sw_ref, w13_ref, w2_ref, gua_ref, gub_ref,
              da_ref, db_ref, out_ref):
    i = pl.program_id(0)
    active = be_ref[i] >= 0

    @pl.when(active)
    def _():
        bf = jnp.bfloat16
        x = xs_ref[...].astype(bf)
        dn = (((1,), (1,)), ((), ()))
        gate_up = lax.dot_general(x, w13_ref[0].astype(bf), dn,
                                  preferred_element_type=jnp.float32)
        mid = lax.dot_general(x, gua_ref[0].astype(bf), dn,
                              preferred_element_type=jnp.float32)
        gate_up = gate_up + lax.dot_general(mid.astype(bf),
                                            gub_ref[0].astype(bf), dn,
                                            preferred_element_type=jnp.float32)
        gate = gate_up[:, :_I]
        up = gate_up[:, _I:]
        act = (gate * jax.nn.sigmoid(gate) * up).astype(bf)
        down = lax.dot_general(act, w2_ref[0].astype(bf), dn,
                               preferred_element_type=jnp.float32)
        dmid = lax.dot_general(act, da_ref[0].astype(bf), dn,
                               preferred_element_type=jnp.float32)
        down = down + lax.dot_general(dmid.astype(bf), db_ref[0].astype(bf), dn,
                                      preferred_element_type=jnp.float32)
        out_ref[...] = sw_ref[...] * down

    @pl.when(jnp.logical_not(active))
    def _():
        out_ref[...] = jnp.zeros_like(out_ref)


def _run_gmm(be, xs, sw2d, w13, w2, gua, gub, da, db):
    def emap(i, be_r):
        return (jnp.where(be_r[i] < 0, _E - 1, be_r[i]), 0, 0)

    grid_spec = pltpu.PrefetchScalarGridSpec(
        num_scalar_prefetch=1,
        grid=(_NB,),
        in_specs=[
            pl.BlockSpec((_BT, _H), lambda i, be_r: (i, 0)),
            pl.BlockSpec((_BT, 1), lambda i, be_r: (i, 0)),
            pl.BlockSpec((1, 2 * _I, _H), emap),
            pl.BlockSpec((1, _H, _I), emap),
            pl.BlockSpec((1, _R, _H), emap),
            pl.BlockSpec((1, 2 * _I, _R), emap),
            pl.BlockSpec((1, _R, _I), emap),
            pl.BlockSpec((1, _H, _R), emap),
        ],
        out_specs=pl.BlockSpec((_BT, _H), lambda i, be_r: (i, 0)),
    )
    return pl.pallas_call(
        _gmm_body,
        grid_spec=grid_spec,
        out_shape=jax.ShapeDtypeStruct((_P, _H), jnp.float32),
    )(be, xs, sw2d, w13, w2, gua, gub, da, db)


@functools.lru_cache(maxsize=None)
def _make_combine():
    return functools.partial(
        pl.kernel,
        out_type=jax.ShapeDtypeStruct((_T, _H), jnp.float32),
        mesh=plsc.VectorSubcoreMesh(**_MESH),
        compiler_params=pltpu.CompilerParams(needs_layout_passes=False),
        scratch_types=[
            pltpu.VMEM((_TPT,), jnp.int32),
            pltpu.VMEM((_TPT,), jnp.int32),
            pltpu.VMEM((_CCH, _H), jnp.float32),
            pltpu.VMEM((_CCH, _H), jnp.float32),
            pltpu.SemaphoreType.DMA,
            pltpu.SemaphoreType.DMA,
        ],
    )(_combine_body)


def _combine_body(dstT_hbm, ys_hbm, out_hbm, idxA_v, idxB_v, bufA, bufB,
                  semA, semB):
    wid = lax.axis_index("s") * 2 + lax.axis_index("c")
    tbase = wid * _TPT
    pltpu.sync_copy(dstT_hbm.at[0, pl.ds(tbase, _TPT)], idxA_v)
    pltpu.sync_copy(dstT_hbm.at[1, pl.ds(tbase, _TPT)], idxB_v)

    for c2 in range(_TPT // _CCH):
        cpA = pltpu.async_copy(ys_hbm.at[idxA_v.at[pl.ds(c2 * _CCH, _CCH)]],
                               bufA, semA)
        cpB = pltpu.async_copy(ys_hbm.at[idxB_v.at[pl.ds(c2 * _CCH, _CCH)]],
                               bufB, semB)
        cpA.wait()
        cpB.wait()

        def add_body(j, _):
            r = j >> 6
            c = j & 63
            plsc.addupdate(bufA.at[r, pl.ds(c * 16, 16)],
                           bufB[r, pl.ds(c * 16, 16)])
            return 0
        lax.fori_loop(0, _CCH * (_H // 16), add_body, 0)
        pltpu.sync_copy(bufA, out_hbm.at[pl.ds(tbase + c2 * _CCH, _CCH)])


@jax.jit
def kernel(hidden_states, topk_weights, topk_ids, w13, w2, gate_up_lora_a,
           gate_up_lora_b, down_lora_a, down_lora_b):
    dst, tww, be = _run_meta(topk_ids, topk_weights)
    dst_t = dst.T
    tww_t = tww.T
    be24 = be.reshape(-1)
    xs, sw = _make_dispatch()(dst_t, tww_t, hidden_states)
    sw2d = sw.reshape(_P, 1)
    ys = _run_gmm(be24, xs, sw2d, w13, w2, gate_up_lora_a, gate_up_lora_b,
                  down_lora_a, down_lora_b)
    return _make_combine()(dst_t, ys)
